# baseline (speedup 1.0000x reference)
import functools

import jax
import jax.numpy as jnp
from jax import lax
from jax.experimental import pallas as pl
from jax.experimental.pallas import tpu as pltpu
from jax.experimental.pallas import tpu_sc as plsc


def _edge_block_sc(edge_attr, x, sender2, receiver2, *, chunk):
    E, DE = edge_attr.shape
    N, DF = x.shape
    DOUT = DE + 2 * DF

    info = plsc.get_sparse_core_info()
    NC, NS = info.num_cores, info.num_subcores
    NW = NC * NS
    assert E % NW == 0
    epw = E // NW
    assert epw % (2 * chunk) == 0
    n_outer = epw // (2 * chunk)
    spw = epw // chunk

    mesh = plsc.VectorSubcoreMesh(core_axis_name="c", subcore_axis_name="s")

    @functools.partial(
        pl.kernel,
        mesh=mesh,
        compiler_params=pltpu.CompilerParams(use_tc_tiling_on_sc=False),
        out_type=jax.ShapeDtypeStruct((E, DOUT), jnp.float32),
        scratch_types=[
            pltpu.VMEM((E // NW // chunk, chunk), jnp.int32),
            pltpu.VMEM((E // NW // chunk, chunk), jnp.int32),
            pltpu.VMEM((chunk, DE), jnp.float32),
            pltpu.VMEM((chunk, DF), jnp.float32),
            pltpu.VMEM((chunk, DF), jnp.float32),
            pltpu.VMEM_SHARED((NS, chunk, DOUT), jnp.float32),
            pltpu.VMEM_SHARED((NS, chunk, DOUT), jnp.float32),
            pltpu.SemaphoreType.DMA,
            pltpu.SemaphoreType.DMA,
            pltpu.SemaphoreType.DMA,
            pltpu.SemaphoreType.DMA,
        ],
    )
    def k(ea_hbm, x_hbm, snd_hbm, rcv_hbm, out_hbm,
          snd_v, rcv_v, at_v, rr_v, sr_v, sb0, sb1, gs, cs, ws0, ws1):
        wid = lax.axis_index("s") * NC + lax.axis_index("c")
        sid = lax.axis_index("s")
        base0 = wid * epw
        sbs, wss = (sb0, sb1), (ws0, ws1)

        pltpu.sync_copy(snd_hbm.at[pl.ds(wid * spw, spw)], snd_v)
        pltpu.sync_copy(rcv_hbm.at[pl.ds(wid * spw, spw)], rcv_v)

        def drain_write(b):
            pltpu.make_async_copy(
                sbs[b].at[sid], out_hbm.at[pl.ds(base0, chunk)], wss[b]).wait()

        def do_chunk(g, b, first):
            base = base0 + g * chunk
            cp_r = pltpu.async_copy(x_hbm.at[rcv_v.at[g]], rr_v, gs)
            cp_s = pltpu.async_copy(x_hbm.at[snd_v.at[g]], sr_v, gs)
            cp_a = pltpu.async_copy(ea_hbm.at[pl.ds(base, chunk)], at_v, gs)
            cp_r.wait()
            cp_s.wait()
            cp_a.wait()

            @pl.when(jnp.logical_not(first))
            def _():
                drain_write(b)

            cp0 = pltpu.async_copy(at_v, sbs[b].at[sid, :, pl.ds(0, DE)], cs)
            cp1 = pltpu.async_copy(rr_v, sbs[b].at[sid, :, pl.ds(DE, DF)], cs)
            cp2 = pltpu.async_copy(sr_v, sbs[b].at[sid, :, pl.ds(DE + DF, DF)], cs)
            cp0.wait()
            cp1.wait()
            cp2.wait()
            pltpu.async_copy(sbs[b].at[sid], out_hbm.at[pl.ds(base, chunk)], wss[b])

        def outer(i, carry):
            for b in range(2):
                do_chunk(2 * i + b, b, i == 0)
            return carry

        lax.fori_loop(0, n_outer, outer, 0)
        drain_write(0)
        drain_write(1)

    return k(edge_attr, x, sender2, receiver2)


@jax.jit
def kernel(edge_attr, x, edge_index):
    chunk = 125
    sender2 = edge_index[0].reshape(-1, chunk)
    receiver2 = edge_index[1].reshape(-1, chunk)
    return _edge_block_sc(edge_attr, x, sender2, receiver2, chunk=chunk)


# speedup vs baseline: 1.0345x; 1.0345x over previous
import functools

import jax
import jax.numpy as jnp
from jax import lax
from jax.experimental import pallas as pl
from jax.experimental.pallas import tpu as pltpu
from jax.experimental.pallas import tpu_sc as plsc


def _edge_block_sc(edge_attr, x, sender, receiver, *, chunk):
    E, DE = edge_attr.shape
    N, DF = x.shape
    DOUT = DE + 2 * DF

    info = plsc.get_sparse_core_info()
    NC, NS = info.num_cores, info.num_subcores
    NW = NC * NS
    assert E % NW == 0
    epw = E // NW
    assert epw % (2 * chunk) == 0
    n_outer = epw // (2 * chunk)

    mesh = plsc.VectorSubcoreMesh(core_axis_name="c", subcore_axis_name="s")

    @functools.partial(
        pl.kernel,
        mesh=mesh,
        compiler_params=pltpu.CompilerParams(use_tc_tiling_on_sc=False),
        out_type=jax.ShapeDtypeStruct((E, DOUT), jnp.float32),
        scratch_types=[
            pltpu.VMEM((E // 32,), jnp.int32),
            pltpu.VMEM((E // 32,), jnp.int32),
            pltpu.VMEM((chunk, DE), jnp.float32),
            pltpu.VMEM((chunk, DE), jnp.float32),
            pltpu.VMEM((chunk, DF), jnp.float32),
            pltpu.VMEM((chunk, DF), jnp.float32),
            pltpu.VMEM((chunk, DF), jnp.float32),
            pltpu.VMEM((chunk, DF), jnp.float32),
            pltpu.SemaphoreType.DMA,
            pltpu.SemaphoreType.DMA,
            pltpu.SemaphoreType.DMA,
            pltpu.SemaphoreType.DMA,
        ],
    )
    def k(ea_hbm, x_hbm, snd_hbm, rcv_hbm, out_hbm,
          snd_v, rcv_v, a0, a1, r0, r1, s0, s1, gs0, gs1, ws0, ws1):
        wid = lax.axis_index("s") * NC + lax.axis_index("c")
        base0 = wid * epw
        ats, rrs, srs = (a0, a1), (r0, r1), (s0, s1)
        gss, wss = (gs0, gs1), (ws0, ws1)

        pltpu.sync_copy(snd_hbm.at[pl.ds(base0, epw)], snd_v)
        pltpu.sync_copy(rcv_hbm.at[pl.ds(base0, epw)], rcv_v)

        def drain_writes(b):
            pltpu.make_async_copy(
                ats[b], out_hbm.at[pl.ds(base0, chunk), pl.ds(0, DE)], wss[b]).wait()
            pltpu.make_async_copy(
                rrs[b], out_hbm.at[pl.ds(base0, chunk), pl.ds(DE, DF)], wss[b]).wait()
            pltpu.make_async_copy(
                srs[b], out_hbm.at[pl.ds(base0, chunk), pl.ds(DE + DF, DF)], wss[b]).wait()

        def drain_gathers(b):
            h = 96
            pltpu.make_async_copy(
                x_hbm.at[rcv_v.at[pl.ds(0, h)]], rrs[b].at[pl.ds(0, h)], gss[b]).wait()
            pltpu.make_async_copy(
                x_hbm.at[rcv_v.at[pl.ds(0, h)]], srs[b].at[pl.ds(0, h)], gss[b]).wait()
            pltpu.make_async_copy(
                x_hbm.at[rcv_v.at[pl.ds(0, chunk - h)]],
                rrs[b].at[pl.ds(h, chunk - h)], gss[b]).wait()
            pltpu.make_async_copy(
                x_hbm.at[rcv_v.at[pl.ds(0, chunk - h)]],
                srs[b].at[pl.ds(h, chunk - h)], gss[b]).wait()
            pltpu.make_async_copy(ea_hbm.at[pl.ds(base0, chunk)], ats[b], gss[b]).wait()

        def outer(i, carry):
            @pl.when(i > 0)
            def _():
                drain_writes(0)
                drain_writes(1)

            for b in range(2):
                g = 2 * i + b
                base = base0 + g * chunk
                off = g * chunk
                h = 96
                pltpu.async_copy(
                    x_hbm.at[rcv_v.at[pl.ds(off, h)]], rrs[b].at[pl.ds(0, h)], gss[b])
                pltpu.async_copy(
                    x_hbm.at[snd_v.at[pl.ds(off, h)]], srs[b].at[pl.ds(0, h)], gss[b])
                pltpu.async_copy(
                    x_hbm.at[rcv_v.at[pl.ds(off + h, chunk - h)]],
                    rrs[b].at[pl.ds(h, chunk - h)], gss[b])
                pltpu.async_copy(
                    x_hbm.at[snd_v.at[pl.ds(off + h, chunk - h)]],
                    srs[b].at[pl.ds(h, chunk - h)], gss[b])
                pltpu.async_copy(ea_hbm.at[pl.ds(base, chunk)], ats[b], gss[b])

            for b in range(2):
                base = base0 + (2 * i + b) * chunk
                drain_gathers(b)
                pltpu.async_copy(
                    ats[b], out_hbm.at[pl.ds(base, chunk), pl.ds(0, DE)], wss[b])
                pltpu.async_copy(
                    rrs[b], out_hbm.at[pl.ds(base, chunk), pl.ds(DE, DF)], wss[b])
                pltpu.async_copy(
                    srs[b], out_hbm.at[pl.ds(base, chunk), pl.ds(DE + DF, DF)], wss[b])
            return carry

        lax.fori_loop(0, n_outer, outer, 0)
        drain_writes(0)
        drain_writes(1)

    return k(edge_attr, x, sender, receiver)


@jax.jit
def kernel(edge_attr, x, edge_index):
    sender = edge_index[0]
    receiver = edge_index[1]
    return _edge_block_sc(edge_attr, x, sender, receiver, chunk=200)
